# baseline (speedup 1.0000x reference)
import jax
import jax.numpy as jnp
from jax import lax
from jax.experimental import pallas as pl
from jax.experimental.pallas import tpu as pltpu
from jax.experimental.pallas import tpu_sc as plsc

N = 10000
K = 32
D = 128
TOTAL = N * K
NC = 2
NS = 16
NW = NC * NS
PER_W = TOTAL // NW
CH = 80
NCH = PER_W // CH
NBUF = 3
NGRP = -(-NCH // NBUF)
HFRAC = 10
STAGE = 632


def _gather_body(table_hbm, idx_hbm, out_hbm, tab_sp, idx_v, rows_v, *sems):
    gsems = sems[:NBUF]
    osems = sems[NBUF:]
    sid = lax.axis_index("s")
    wid = sid * NC + lax.axis_index("c")
    base = wid * PER_W
    pltpu.sync_copy(idx_hbm.at[wid], idx_v)
    for s in range(NS):
        lo = s * STAGE
        span = min(STAGE, N - lo)

        @pl.when(sid == s)
        def _():
            pltpu.sync_copy(
                table_hbm.at[pl.ds(lo, span)], tab_sp.at[pl.ds(lo, span)]
            )

    plsc.subcore_barrier()

    def g_start(j, b):
        @pl.when(j % HFRAC == 0)
        def _():
            pltpu.make_async_copy(
                table_hbm.at[idx_v.at[j]], rows_v.at[b], gsems[b]
            ).start()

        @pl.when(j % HFRAC != 0)
        def _():
            pltpu.make_async_copy(
                tab_sp.at[idx_v.at[j]], rows_v.at[b], gsems[b]
            ).start()

    def g_wait(j, b):
        @pl.when(j % HFRAC == 0)
        def _():
            pltpu.make_async_copy(
                table_hbm.at[idx_v.at[j]], rows_v.at[b], gsems[b]
            ).wait()

        @pl.when(j % HFRAC != 0)
        def _():
            pltpu.make_async_copy(
                tab_sp.at[idx_v.at[j]], rows_v.at[b], gsems[b]
            ).wait()

    def o_desc(j, b):
        return pltpu.make_async_copy(
            rows_v.at[b], out_hbm.at[pl.ds(base + j * CH, CH)], osems[b]
        )

    for b in range(NBUF):
        g_start(b, b)

    def outer(g, carry):
        j0 = g * NBUF
        for b in range(NBUF):
            j = j0 + b

            @pl.when(j < NCH)
            def _():
                g_wait(j, b)
                o_desc(j, b).start()

        for b in range(NBUF):
            nxt = j0 + b + NBUF

            @pl.when(nxt < NCH)
            def _():
                o_desc(j0 + b, b).wait()
                g_start(nxt, b)
        return carry

    lax.fori_loop(0, NGRP, outer, 0)
    for j in range(NCH - NBUF, NCH):
        o_desc(j, j % NBUF).wait()


@jax.jit
def kernel(x_features, x_graph):
    table = x_features.reshape(N, D)
    idx = x_graph.reshape(NW, NCH, CH)
    mesh = plsc.VectorSubcoreMesh(
        core_axis_name="c", subcore_axis_name="s", num_cores=NC, num_subcores=NS
    )
    out = pl.kernel(
        _gather_body,
        out_type=jax.ShapeDtypeStruct((TOTAL, D), jnp.float32),
        mesh=mesh,
        scratch_types=(
            [
                pltpu.VMEM_SHARED((N, D), jnp.float32),
                pltpu.VMEM((NCH, CH), jnp.int32),
                pltpu.VMEM((NBUF, CH, D), jnp.float32),
            ]
            + [pltpu.SemaphoreType.DMA] * (2 * NBUF)
        ),
    )(table, idx)
    return out.reshape(1, N, K, D)


# speedup vs baseline: 1.1881x; 1.1881x over previous
import jax
import jax.numpy as jnp
from jax import lax
from jax.experimental import pallas as pl
from jax.experimental.pallas import tpu as pltpu
from jax.experimental.pallas import tpu_sc as plsc

N = 10000
K = 32
D = 128
TOTAL = N * K
NC = 2
NS = 16
NW = NC * NS
PER_W = TOTAL // NW
CH = 80
NCH = PER_W // CH
NBUF = 3
NGRP = -(-NCH // NBUF)
STAGE = 632


def _gather_body(table_hbm, idx_hbm, out_hbm, tab_sp, idx_v, rows_v, *sems):
    gsems = sems[:NBUF]
    osems = sems[NBUF:2 * NBUF]
    ssem = sems[2 * NBUF]
    sid = lax.axis_index("s")
    wid = sid * NC + lax.axis_index("c")
    base = wid * PER_W
    stage_descs = []
    for s in range(NS):
        lo = s * STAGE
        span = min(STAGE, N - lo)

        @pl.when(sid == s)
        def _():
            pltpu.make_async_copy(
                table_hbm.at[pl.ds(lo, span)], tab_sp.at[pl.ds(lo, span)], ssem
            ).start()

        stage_descs.append((lo, span))
    pltpu.sync_copy(idx_hbm.at[wid], idx_v)

    def g_desc(j, b, src):
        return pltpu.make_async_copy(
            src.at[idx_v.at[j]], rows_v.at[b], gsems[b]
        )

    def o_desc(j, b):
        return pltpu.make_async_copy(
            rows_v.at[b], out_hbm.at[pl.ds(base + j * CH, CH)], osems[b]
        )

    for b in range(NBUF):
        g_desc(b, b, table_hbm).start()
    for s in range(NS):
        lo, span = stage_descs[s]

        @pl.when(sid == s)
        def _():
            pltpu.make_async_copy(
                table_hbm.at[pl.ds(lo, span)], tab_sp.at[pl.ds(lo, span)], ssem
            ).wait()

    plsc.subcore_barrier()

    def outer(g, carry):
        j0 = g * NBUF
        for b in range(NBUF):
            j = j0 + b

            @pl.when(j < NCH)
            def _():
                @pl.when(j < NBUF)
                def _():
                    g_desc(j, b, table_hbm).wait()

                @pl.when(j >= NBUF)
                def _():
                    g_desc(j, b, tab_sp).wait()

                o_desc(j, b).start()

        for b in range(NBUF):
            nxt = j0 + b + NBUF

            @pl.when(nxt < NCH)
            def _():
                o_desc(j0 + b, b).wait()
                g_desc(nxt, b, tab_sp).start()
        return carry

    lax.fori_loop(0, NGRP, outer, 0)
    for j in range(NCH - NBUF, NCH):
        o_desc(j, j % NBUF).wait()


@jax.jit
def kernel(x_features, x_graph):
    table = x_features.reshape(N, D)
    idx = x_graph.reshape(NW, NCH, CH)
    mesh = plsc.VectorSubcoreMesh(
        core_axis_name="c", subcore_axis_name="s", num_cores=NC, num_subcores=NS
    )
    out = pl.kernel(
        _gather_body,
        out_type=jax.ShapeDtypeStruct((TOTAL, D), jnp.float32),
        mesh=mesh,
        scratch_types=(
            [
                pltpu.VMEM_SHARED((N, D), jnp.float32),
                pltpu.VMEM((NCH, CH), jnp.int32),
                pltpu.VMEM((NBUF, CH, D), jnp.float32),
            ]
            + [pltpu.SemaphoreType.DMA] * (2 * NBUF + 1)
        ),
    )(table, idx)
    return out.reshape(1, N, K, D)
